# baseline (speedup 1.0000x reference)
import functools

import jax
import jax.numpy as jnp
from jax import lax
from jax.experimental import pallas as pl
from jax.experimental.pallas import tpu as pltpu
from jax.experimental.pallas import tpu_sc as plsc
from jax._src.pallas import mpmd as _mpmd

D = 1024
V = 65536
B = 8192
BM = 512
KC = 2048
NC, NS = 2, 16
NW = NC * NS
BPW = B // NW
CH = 32
NCHW = BPW // CH

_MESH = plsc.VectorSubcoreMesh(
    core_axis_name="c", subcore_axis_name="s", num_cores=NC, num_subcores=NS
)

_SC_SCRATCH = [
    pltpu.VMEM((NCHW, CH), jnp.int32),
    pltpu.VMEM((CH, D), jnp.float32),
    pltpu.VMEM((CH, D), jnp.float32),
    pltpu.SemaphoreType.DMA,
    pltpu.SemaphoreType.DMA,
]


def _worker_id():
    return lax.axis_index("s") * NC + lax.axis_index("c")


def _gather_body(mem_h, idx_h, out_h, idx_v, buf0, buf1, sem0, sem1):
    wid = _worker_id()
    pltpu.sync_copy(idx_h.at[pl.ds(wid * NCHW, NCHW)], idx_v)
    bufs, sems = (buf0, buf1), (sem0, sem1)
    descs = [None, None]
    descs[0] = pltpu.async_copy(mem_h.at[idx_v.at[0]], bufs[0], sems[0])
    for ci in range(NCHW):
        if ci + 1 < NCHW:
            nb = (ci + 1) % 2
            descs[nb] = pltpu.async_copy(mem_h.at[idx_v.at[ci + 1]], bufs[nb], sems[nb])
        descs[ci % 2].wait()
        pltpu.sync_copy(bufs[ci % 2], out_h.at[pl.ds(wid * BPW + ci * CH, CH)])


_gather = pl.kernel(
    _gather_body,
    out_type=jax.ShapeDtypeStruct((B, D), jnp.float32),
    mesh=_MESH,
    scratch_types=_SC_SCRATCH,
    name="hfa_sc_gather",
)


def _scatter_body(mem_h, idx_h, wv_h, out_h, idx_v, buf0, buf1, sem0, sem1):
    del mem_h
    wid = _worker_id()
    pltpu.sync_copy(idx_h.at[pl.ds(wid * NCHW, NCHW)], idx_v)
    bufs, sems = (buf0, buf1), (sem0, sem1)
    descs = [None, None]
    descs[0] = pltpu.async_copy(wv_h.at[pl.ds(wid * BPW, CH)], bufs[0], sems[0])
    for ci in range(NCHW):
        if ci + 1 < NCHW:
            nb = (ci + 1) % 2
            descs[nb] = pltpu.async_copy(
                wv_h.at[pl.ds(wid * BPW + (ci + 1) * CH, CH)], bufs[nb], sems[nb]
            )
        descs[ci % 2].wait()
        pltpu.sync_copy(bufs[ci % 2], out_h.at[idx_v.at[ci]])


_scatter = _mpmd._mpmd_map(
    [(_MESH, _scatter_body)],
    out_types=jax.ShapeDtypeStruct((V, D), jnp.float32),
    input_output_aliases={0: 0},
    scratch_types=_SC_SCRATCH,
    name="hfa_sc_scatter",
)


_VPM = V // (B // BM)


def _mlp_body(mem_ref, val_ref, prev_ref, w1v_ref, w1p_ref, b1_ref, w2t_ref,
              b2_ref, fp_ref, fv_ref, fg_ref, bf1_ref, wf2_ref, bf2_ref,
              out_ref, base_ref, copy_sem):
    m = pl.program_id(0)
    nm = pl.num_programs(0)
    pltpu.make_async_copy(
        mem_ref.at[pl.ds(m * _VPM, _VPM)],
        base_ref.at[pl.ds(m * _VPM, _VPM)],
        copy_sem,
    ).start()

    @pl.when(m == nm - 1)
    def _drain():
        for _ in range(B // BM):
            pltpu.make_async_copy(
                mem_ref.at[pl.ds(0, _VPM)],
                base_ref.at[pl.ds(0, _VPM)],
                copy_sem,
            ).wait()

    xv = val_ref[...]
    xp = prev_ref[...]
    xv16 = xv.astype(jnp.bfloat16)
    xp16 = xp.astype(jnp.bfloat16)
    h = jnp.maximum(
        jnp.dot(xv16, w1v_ref[...], preferred_element_type=jnp.float32)
        + jnp.dot(xp16, w1p_ref[...], preferred_element_type=jnp.float32)
        + b1_ref[...],
        0.0,
    )
    glogit = jnp.sum(h * w2t_ref[...], axis=1, keepdims=True) + b2_ref[0, 0]
    gate = jax.nn.sigmoid(glogit)
    pg16 = (xp * gate).astype(jnp.bfloat16)
    u = jnp.maximum(
        jnp.dot(xp16, fp_ref[...], preferred_element_type=jnp.float32)
        + jnp.dot(xv16, fv_ref[...], preferred_element_type=jnp.float32)
        + jnp.dot(pg16, fg_ref[...], preferred_element_type=jnp.float32)
        + bf1_ref[...],
        0.0,
    )
    interp = jnp.tanh(
        jnp.dot(u.astype(jnp.bfloat16), wf2_ref[...],
                preferred_element_type=jnp.float32)
        + bf2_ref[...]
    )
    out_ref[...] = (gate * (interp - xp)).astype(jnp.bfloat16)


def _const2(i, j):
    return lambda m: (i, j)


_mlp = pl.pallas_call(
    _mlp_body,
    grid=(B // BM,),
    in_specs=[
        pl.BlockSpec(memory_space=pltpu.HBM),
        pl.BlockSpec((BM, D), lambda m: (m, 0)),
        pl.BlockSpec((BM, D), lambda m: (m, 0)),
        pl.BlockSpec((D, D), _const2(0, 0)),
        pl.BlockSpec((D, D), _const2(0, 0)),
        pl.BlockSpec((1, D), _const2(0, 0)),
        pl.BlockSpec((1, D), _const2(0, 0)),
        pl.BlockSpec((1, 128), _const2(0, 0)),
        pl.BlockSpec((D, 2 * D), _const2(0, 0)),
        pl.BlockSpec((D, 2 * D), _const2(0, 0)),
        pl.BlockSpec((D, 2 * D), _const2(0, 0)),
        pl.BlockSpec((1, 2 * D), _const2(0, 0)),
        pl.BlockSpec((2 * D, D), _const2(0, 0)),
        pl.BlockSpec((1, D), _const2(0, 0)),
    ],
    out_specs=[
        pl.BlockSpec((BM, D), lambda m: (m, 0)),
        pl.BlockSpec(memory_space=pltpu.HBM),
    ],
    out_shape=[
        jax.ShapeDtypeStruct((B, D), jnp.bfloat16),
        jax.ShapeDtypeStruct((V, D), jnp.float32),
    ],
    scratch_shapes=[pltpu.SemaphoreType.DMA],
    name="hfa_tc_mlp",
)


def _combine_body(idxc_ref, idxr_ref, d16_ref, prev_ref, out_ref):
    me = idxc_ref[:, 0:1]
    acc = jnp.zeros((BM, D), jnp.float32)
    for c in range(B // KC):
        ks = idxr_ref[0, :, pl.ds(c * KC, KC)]
        a = (me == ks).astype(jnp.bfloat16)
        acc = acc + jnp.dot(a, d16_ref[pl.ds(c * KC, KC), :],
                            preferred_element_type=jnp.float32)
    out_ref[...] = prev_ref[...] + acc


_combine = pl.pallas_call(
    _combine_body,
    grid=(B // BM,),
    in_specs=[
        pl.BlockSpec((BM, 128), lambda m: (m, 0)),
        pl.BlockSpec((1, 1, B), lambda m: (0, 0, 0)),
        pl.BlockSpec((B, D), _const2(0, 0)),
        pl.BlockSpec((BM, D), lambda m: (m, 0)),
    ],
    out_specs=pl.BlockSpec((BM, D), lambda m: (m, 0)),
    out_shape=jax.ShapeDtypeStruct((B, D), jnp.float32),
    name="hfa_tc_combine",
)


def kernel(mem, idx, val, W_sd1, b_sd1, W_sd2, b_sd2, W_fi1, b_fi1, W_fi2, b_fi2):
    idx32 = idx.astype(jnp.int32)
    idx2 = idx32.reshape(B // CH, CH)

    prev = _gather(mem, idx2)

    bf16 = jnp.bfloat16
    delta16, base = _mlp(
        mem, val, prev,
        W_sd1[:D].astype(bf16), W_sd1[D:].astype(bf16),
        b_sd1.reshape(1, D),
        W_sd2.reshape(1, D),
        jnp.broadcast_to(b_sd2.reshape(1, 1), (1, 128)),
        W_fi1[:D].astype(bf16), W_fi1[D:2 * D].astype(bf16),
        W_fi1[2 * D:].astype(bf16),
        b_fi1.reshape(1, 2 * D),
        W_fi2.astype(bf16),
        b_fi2.reshape(1, D),
    )

    idx_mcol = jnp.broadcast_to(idx32[:, None], (B, 128))
    idx_row3 = idx32.reshape(1, 1, B)
    wv = _combine(idx_mcol, idx_row3, delta16, prev)

    return _scatter(base, idx2, wv)


# speedup vs baseline: 1.0212x; 1.0212x over previous
import functools

import jax
import jax.numpy as jnp
from jax import lax
from jax.experimental import pallas as pl
from jax.experimental.pallas import tpu as pltpu
from jax.experimental.pallas import tpu_sc as plsc
from jax._src.pallas import mpmd as _mpmd

D = 1024
V = 65536
B = 8192
BM = 512
KC = 2048
NC, NS = 2, 16
NW = NC * NS
BPW = B // NW
CH = 32
NCHW = BPW // CH

_MESH = plsc.VectorSubcoreMesh(
    core_axis_name="c", subcore_axis_name="s", num_cores=NC, num_subcores=NS
)

_SC_SCRATCH = [
    pltpu.VMEM((NCHW, CH), jnp.int32),
    pltpu.VMEM((CH, D), jnp.float32),
    pltpu.VMEM((CH, D), jnp.float32),
    pltpu.SemaphoreType.DMA,
    pltpu.SemaphoreType.DMA,
]


def _worker_id():
    return lax.axis_index("s") * NC + lax.axis_index("c")


def _gather_body(mem_h, idx_h, out_h, idx_v, buf0, buf1, sem0, sem1):
    wid = _worker_id()
    pltpu.sync_copy(idx_h.at[pl.ds(wid * NCHW, NCHW)], idx_v)
    bufs, sems = (buf0, buf1), (sem0, sem1)
    descs = [None, None]
    descs[0] = pltpu.async_copy(mem_h.at[idx_v.at[0]], bufs[0], sems[0])
    for ci in range(NCHW):
        if ci + 1 < NCHW:
            nb = (ci + 1) % 2
            descs[nb] = pltpu.async_copy(mem_h.at[idx_v.at[ci + 1]], bufs[nb], sems[nb])
        descs[ci % 2].wait()
        pltpu.sync_copy(bufs[ci % 2], out_h.at[pl.ds(wid * BPW + ci * CH, CH)])


_gather = pl.kernel(
    _gather_body,
    out_type=jax.ShapeDtypeStruct((B, D), jnp.float32),
    mesh=_MESH,
    scratch_types=_SC_SCRATCH,
    name="hfa_sc_gather",
)


def _scatter_body(mem_h, idx_h, wv_h, out_h, idx_v, buf0, buf1, sem0, sem1):
    del mem_h
    wid = _worker_id()
    pltpu.sync_copy(idx_h.at[pl.ds(wid * NCHW, NCHW)], idx_v)
    bufs, sems = (buf0, buf1), (sem0, sem1)
    descs = [None, None]
    descs[0] = pltpu.async_copy(wv_h.at[pl.ds(wid * BPW, CH)], bufs[0], sems[0])
    for ci in range(NCHW):
        if ci + 1 < NCHW:
            nb = (ci + 1) % 2
            descs[nb] = pltpu.async_copy(
                wv_h.at[pl.ds(wid * BPW + (ci + 1) * CH, CH)], bufs[nb], sems[nb]
            )
        descs[ci % 2].wait()
        pltpu.sync_copy(bufs[ci % 2], out_h.at[idx_v.at[ci]])


_scatter = _mpmd._mpmd_map(
    [(_MESH, _scatter_body)],
    out_types=jax.ShapeDtypeStruct((V, D), jnp.float32),
    input_output_aliases={0: 0},
    scratch_types=_SC_SCRATCH,
    name="hfa_sc_scatter",
)


_VPW = V // NW


def _copy_body(mem_h, out_h):
    wid = _worker_id()
    pltpu.sync_copy(
        mem_h.at[pl.ds(wid * _VPW, _VPW)], out_h.at[pl.ds(wid * _VPW, _VPW)]
    )


_copy = pl.kernel(
    _copy_body,
    out_type=jax.ShapeDtypeStruct((V, D), jnp.float32),
    mesh=_MESH,
    name="hfa_sc_copy",
)


def _mlp_body(val_ref, prev_ref, w1v_ref, w1p_ref, b1_ref, w2t_ref,
              b2_ref, fp_ref, fv_ref, fg_ref, bf1_ref, wf2_ref, bf2_ref,
              out_ref):
    xv = val_ref[...]
    xp = prev_ref[...]
    xv16 = xv.astype(jnp.bfloat16)
    xp16 = xp.astype(jnp.bfloat16)
    h = jnp.maximum(
        jnp.dot(xv16, w1v_ref[...], preferred_element_type=jnp.float32)
        + jnp.dot(xp16, w1p_ref[...], preferred_element_type=jnp.float32)
        + b1_ref[...],
        0.0,
    )
    glogit = jnp.sum(h * w2t_ref[...], axis=1, keepdims=True) + b2_ref[0, 0]
    gate = jax.nn.sigmoid(glogit)
    pg16 = (xp * gate).astype(jnp.bfloat16)
    u = jnp.maximum(
        jnp.dot(xp16, fp_ref[...], preferred_element_type=jnp.float32)
        + jnp.dot(xv16, fv_ref[...], preferred_element_type=jnp.float32)
        + jnp.dot(pg16, fg_ref[...], preferred_element_type=jnp.float32)
        + bf1_ref[...],
        0.0,
    )
    interp = jnp.tanh(
        jnp.dot(u.astype(jnp.bfloat16), wf2_ref[...],
                preferred_element_type=jnp.float32)
        + bf2_ref[...]
    )
    out_ref[...] = (gate * (interp - xp)).astype(jnp.bfloat16)


def _const2(i, j):
    return lambda m: (i, j)


_mlp = pl.pallas_call(
    _mlp_body,
    grid=(B // BM,),
    in_specs=[
        pl.BlockSpec((BM, D), lambda m: (m, 0)),
        pl.BlockSpec((BM, D), lambda m: (m, 0)),
        pl.BlockSpec((D, D), _const2(0, 0)),
        pl.BlockSpec((D, D), _const2(0, 0)),
        pl.BlockSpec((1, D), _const2(0, 0)),
        pl.BlockSpec((1, D), _const2(0, 0)),
        pl.BlockSpec((1, 128), _const2(0, 0)),
        pl.BlockSpec((D, 2 * D), _const2(0, 0)),
        pl.BlockSpec((D, 2 * D), _const2(0, 0)),
        pl.BlockSpec((D, 2 * D), _const2(0, 0)),
        pl.BlockSpec((1, 2 * D), _const2(0, 0)),
        pl.BlockSpec((2 * D, D), _const2(0, 0)),
        pl.BlockSpec((1, D), _const2(0, 0)),
    ],
    out_specs=pl.BlockSpec((BM, D), lambda m: (m, 0)),
    out_shape=jax.ShapeDtypeStruct((B, D), jnp.bfloat16),
    name="hfa_tc_mlp",
)


def _combine_body(idxc_ref, idxr_ref, d16_ref, prev_ref, out_ref):
    me = idxc_ref[:, 0:1]
    acc = jnp.zeros((BM, D), jnp.float32)
    for c in range(B // KC):
        ks = idxr_ref[0, :, pl.ds(c * KC, KC)]
        a = (me == ks).astype(jnp.bfloat16)
        acc = acc + jnp.dot(a, d16_ref[pl.ds(c * KC, KC), :],
                            preferred_element_type=jnp.float32)
    out_ref[...] = prev_ref[...] + acc


_combine = pl.pallas_call(
    _combine_body,
    grid=(B // BM,),
    in_specs=[
        pl.BlockSpec((BM, 128), lambda m: (m, 0)),
        pl.BlockSpec((1, 1, B), lambda m: (0, 0, 0)),
        pl.BlockSpec((B, D), _const2(0, 0)),
        pl.BlockSpec((BM, D), lambda m: (m, 0)),
    ],
    out_specs=pl.BlockSpec((BM, D), lambda m: (m, 0)),
    out_shape=jax.ShapeDtypeStruct((B, D), jnp.float32),
    name="hfa_tc_combine",
)


def kernel(mem, idx, val, W_sd1, b_sd1, W_sd2, b_sd2, W_fi1, b_fi1, W_fi2, b_fi2):
    idx32 = idx.astype(jnp.int32)
    idx2 = idx32.reshape(B // CH, CH)

    prev = _gather(mem, idx2)

    bf16 = jnp.bfloat16
    base = _copy(mem)
    delta16 = _mlp(
        val, prev,
        W_sd1[:D].astype(bf16), W_sd1[D:].astype(bf16),
        b_sd1.reshape(1, D),
        W_sd2.reshape(1, D),
        jnp.broadcast_to(b_sd2.reshape(1, 1), (1, 128)),
        W_fi1[:D].astype(bf16), W_fi1[D:2 * D].astype(bf16),
        W_fi1[2 * D:].astype(bf16),
        b_fi1.reshape(1, 2 * D),
        W_fi2.astype(bf16),
        b_fi2.reshape(1, D),
    )

    idx_mcol = jnp.broadcast_to(idx32[:, None], (B, 128))
    idx_row3 = idx32.reshape(1, 1, B)
    wv = _combine(idx_mcol, idx_row3, delta16, prev)

    return _scatter(base, idx2, wv)


# speedup vs baseline: 15.7879x; 15.4608x over previous
import functools

import jax
import jax.numpy as jnp
from jax import lax
from jax.experimental import pallas as pl
from jax.experimental.pallas import tpu as pltpu
from jax.experimental.pallas import tpu_sc as plsc
from jax._src.pallas import mpmd as _mpmd

D = 1024
V = 65536
B = 8192
BM = 512
KC = 2048
NC, NS = 2, 16
NW = NC * NS
BPW = B // NW
CH = 32
NCHW = BPW // CH

_MESH = plsc.VectorSubcoreMesh(
    core_axis_name="c", subcore_axis_name="s", num_cores=NC, num_subcores=NS
)

_SC_SCRATCH = [
    pltpu.VMEM((NCHW, CH), jnp.int32),
    pltpu.VMEM((CH, D), jnp.float32),
    pltpu.VMEM((CH, D), jnp.float32),
    pltpu.SemaphoreType.DMA,
    pltpu.SemaphoreType.DMA,
]


def _worker_id():
    return lax.axis_index("s") * NC + lax.axis_index("c")


def _gather_body(mem_h, idx_h, out_h, idx_v, buf0, buf1, sem0, sem1):
    wid = _worker_id()
    pltpu.sync_copy(idx_h.at[pl.ds(wid * NCHW, NCHW)], idx_v)
    bufs, sems = (buf0, buf1), (sem0, sem1)
    descs = [None, None]
    descs[0] = pltpu.async_copy(mem_h.at[idx_v.at[0]], bufs[0], sems[0])
    for ci in range(NCHW):
        if ci + 1 < NCHW:
            nb = (ci + 1) % 2
            descs[nb] = pltpu.async_copy(mem_h.at[idx_v.at[ci + 1]], bufs[nb], sems[nb])
        descs[ci % 2].wait()
        pltpu.sync_copy(bufs[ci % 2], out_h.at[pl.ds(wid * BPW + ci * CH, CH)])


_gather = pl.kernel(
    _gather_body,
    out_type=jax.ShapeDtypeStruct((B, D), jnp.float32),
    mesh=_MESH,
    scratch_types=_SC_SCRATCH,
    name="hfa_sc_gather",
)


def _scatter_body(mem_h, idx_h, wv_h, out_h, idx_v, buf0, buf1, sem0, sem1):
    del mem_h
    wid = _worker_id()
    pltpu.sync_copy(idx_h.at[pl.ds(wid * NCHW, NCHW)], idx_v)
    bufs, sems = (buf0, buf1), (sem0, sem1)
    descs = [None, None]
    descs[0] = pltpu.async_copy(wv_h.at[pl.ds(wid * BPW, CH)], bufs[0], sems[0])
    for ci in range(NCHW):
        if ci + 1 < NCHW:
            nb = (ci + 1) % 2
            descs[nb] = pltpu.async_copy(
                wv_h.at[pl.ds(wid * BPW + (ci + 1) * CH, CH)], bufs[nb], sems[nb]
            )
        descs[ci % 2].wait()
        pltpu.sync_copy(bufs[ci % 2], out_h.at[idx_v.at[ci]])


_scatter = _mpmd._mpmd_map(
    [(_MESH, _scatter_body)],
    out_types=jax.ShapeDtypeStruct((V, D), jnp.float32),
    input_output_aliases={0: 0},
    scratch_types=_SC_SCRATCH,
    name="hfa_sc_scatter",
)


def _copy_body(mem_ref, out_ref):
    out_ref[...] = mem_ref[...]


_copy = pl.pallas_call(
    _copy_body,
    grid=(16,),
    in_specs=[pl.BlockSpec((2048, D), lambda m: (m + 16, 0))],
    out_specs=pl.BlockSpec((2048, D), lambda m: (m + 16, 0)),
    out_shape=jax.ShapeDtypeStruct((V, D), jnp.float32),
    name="hfa_tc_basecopy",
)


def _mlp_body(val_ref, prev_ref, w1v_ref, w1p_ref, b1_ref, w2t_ref,
              b2_ref, fp_ref, fv_ref, fg_ref, bf1_ref, wf2_ref, bf2_ref,
              memblk_ref, base_in_ref, out_ref, base_ref):
    del base_in_ref
    base_ref[...] = memblk_ref[...]
    xv = val_ref[...]
    xp = prev_ref[...]
    xv16 = xv.astype(jnp.bfloat16)
    xp16 = xp.astype(jnp.bfloat16)
    h = jnp.maximum(
        jnp.dot(xv16, w1v_ref[...], preferred_element_type=jnp.float32)
        + jnp.dot(xp16, w1p_ref[...], preferred_element_type=jnp.float32)
        + b1_ref[...],
        0.0,
    )
    glogit = jnp.sum(h * w2t_ref[...], axis=1, keepdims=True) + b2_ref[0, 0]
    gate = jax.nn.sigmoid(glogit)
    pg16 = (xp * gate).astype(jnp.bfloat16)
    u = jnp.maximum(
        jnp.dot(xp16, fp_ref[...], preferred_element_type=jnp.float32)
        + jnp.dot(xv16, fv_ref[...], preferred_element_type=jnp.float32)
        + jnp.dot(pg16, fg_ref[...], preferred_element_type=jnp.float32)
        + bf1_ref[...],
        0.0,
    )
    interp = jnp.tanh(
        jnp.dot(u.astype(jnp.bfloat16), wf2_ref[...],
                preferred_element_type=jnp.float32)
        + bf2_ref[...]
    )
    out_ref[...] = (gate * (interp - xp)).astype(jnp.bfloat16)


def _const2(i, j):
    return lambda m: (i, j)


_mlp = pl.pallas_call(
    _mlp_body,
    grid=(B // BM,),
    in_specs=[
        pl.BlockSpec((BM, D), lambda m: (m, 0)),
        pl.BlockSpec((BM, D), lambda m: (m, 0)),
        pl.BlockSpec((D, D), _const2(0, 0)),
        pl.BlockSpec((D, D), _const2(0, 0)),
        pl.BlockSpec((1, D), _const2(0, 0)),
        pl.BlockSpec((1, D), _const2(0, 0)),
        pl.BlockSpec((1, 128), _const2(0, 0)),
        pl.BlockSpec((D, 2 * D), _const2(0, 0)),
        pl.BlockSpec((D, 2 * D), _const2(0, 0)),
        pl.BlockSpec((D, 2 * D), _const2(0, 0)),
        pl.BlockSpec((1, 2 * D), _const2(0, 0)),
        pl.BlockSpec((2 * D, D), _const2(0, 0)),
        pl.BlockSpec((1, D), _const2(0, 0)),
        pl.BlockSpec((1024, D), lambda m: (m, 0)),
        pl.BlockSpec(memory_space=pltpu.HBM),
    ],
    out_specs=[
        pl.BlockSpec((BM, D), lambda m: (m, 0)),
        pl.BlockSpec((1024, D), lambda m: (m, 0)),
    ],
    out_shape=[
        jax.ShapeDtypeStruct((B, D), jnp.bfloat16),
        jax.ShapeDtypeStruct((V, D), jnp.float32),
    ],
    input_output_aliases={14: 1},
    name="hfa_tc_mlp",
)


def _combine_body(idxc_ref, idxr_ref, d16_ref, prev_ref, memblk_ref,
                  base_in_ref, out_ref, base_ref):
    del base_in_ref
    base_ref[...] = memblk_ref[...]
    me = idxc_ref[:, 0:1]
    acc = jnp.zeros((BM, D), jnp.float32)
    for c in range(B // KC):
        ks = idxr_ref[0, :, pl.ds(c * KC, KC)]
        a = (me == ks).astype(jnp.bfloat16)
        acc = acc + jnp.dot(a, d16_ref[pl.ds(c * KC, KC), :],
                            preferred_element_type=jnp.float32)
    out_ref[...] = prev_ref[...] + acc


_combine = pl.pallas_call(
    _combine_body,
    grid=(B // BM,),
    in_specs=[
        pl.BlockSpec((BM, 128), lambda m: (m, 0)),
        pl.BlockSpec((1, 1, B), lambda m: (0, 0, 0)),
        pl.BlockSpec((B, D), _const2(0, 0)),
        pl.BlockSpec((BM, D), lambda m: (m, 0)),
        pl.BlockSpec((1024, D), lambda m: (m + 16, 0)),
        pl.BlockSpec(memory_space=pltpu.HBM),
    ],
    out_specs=[
        pl.BlockSpec((BM, D), lambda m: (m, 0)),
        pl.BlockSpec((1024, D), lambda m: (m + 16, 0)),
    ],
    out_shape=[
        jax.ShapeDtypeStruct((B, D), jnp.float32),
        jax.ShapeDtypeStruct((V, D), jnp.float32),
    ],
    input_output_aliases={5: 1},
    name="hfa_tc_combine",
)


def kernel(mem, idx, val, W_sd1, b_sd1, W_sd2, b_sd2, W_fi1, b_fi1, W_fi2, b_fi2):
    idx32 = idx.astype(jnp.int32)
    idx2 = idx32.reshape(B // CH, CH)

    prev = _gather(mem, idx2)

    bf16 = jnp.bfloat16
    base0 = _copy(mem)
    delta16, base1 = _mlp(
        val, prev,
        W_sd1[:D].astype(bf16), W_sd1[D:].astype(bf16),
        b_sd1.reshape(1, D),
        W_sd2.reshape(1, D),
        jnp.broadcast_to(b_sd2.reshape(1, 1), (1, 128)),
        W_fi1[:D].astype(bf16), W_fi1[D:2 * D].astype(bf16),
        W_fi1[2 * D:].astype(bf16),
        b_fi1.reshape(1, 2 * D),
        W_fi2.astype(bf16),
        b_fi2.reshape(1, D),
        mem, base0,
    )

    idx_mcol = jnp.broadcast_to(idx32[:, None], (B, 128))
    idx_row3 = idx32.reshape(1, 1, B)
    wv, base2 = _combine(idx_mcol, idx_row3, delta16, prev, mem, base1)

    return _scatter(base2, idx2, wv)
